# Initial kernel scaffold; baseline (speedup 1.0000x reference)
import jax
import jax.numpy as jnp
from jax.experimental import pallas as pl


def kernel(solute_x, solute_edge_index, solute_edge_attr, solute_batch, solvent_x, solvent_batch, params):
    raise NotImplementedError("write your pallas kernel here")


# speedup vs baseline: 2.0465x; 2.0465x over previous
import functools

import jax
import jax.numpy as jnp
from jax import lax
from jax.experimental import pallas as pl
from jax.experimental.pallas import tpu as pltpu
from jax.experimental.pallas import tpu_sc as plsc

_NC = 2
_NS = 16
_NW = _NC * _NS
_EMB = 128
_NSEG = 256
_NSOLU = 10000
_NSOLV = 5000
_ESOLU = 320000


@functools.cache
def _sc_segsum(n_table, d, e_pad, k, n_out):
    assert e_pad % (_NW * k) == 0 and k % 8 == 0 and k <= 128
    assert n_out % (_NS * 8) == 0
    e_per_w = e_pad // _NW
    n_chunks = e_per_w // k
    stripe = n_out // _NS

    mesh = plsc.VectorSubcoreMesh(core_axis_name="c", subcore_axis_name="s",
                                  num_cores=_NC, num_subcores=_NS)

    @functools.partial(
        pl.kernel,
        out_type=jax.ShapeDtypeStruct((_NC * n_out, d), jnp.float32),
        mesh=mesh,
        scratch_types=[
            pltpu.VMEM((k,), jnp.int32),
            pltpu.VMEM((k,), jnp.int32),
            pltpu.VMEM((k, d), jnp.float32),
            pltpu.VMEM_SHARED((n_out, d), jnp.float32),
            pltpu.SemaphoreType.DMA,
        ],
    )
    def fn(table_hbm, src_hbm, dst_hbm, out_hbm, src_v, dst_v, rows_v,
           acc_sh, sem):
        c = lax.axis_index("c")
        s = lax.axis_index("s")
        w = c * _NS + s

        zero16 = jnp.zeros((16,), jnp.float32)

        def zrow(i, carry):
            for j in range(d // 16):
                rows_v[i, pl.ds(j * 16, 16)] = zero16
            return carry

        lax.fori_loop(0, k, zrow, 0)
        row0 = s * stripe
        off = 0
        while off < stripe:
            step = min(k, stripe - off)
            pltpu.sync_copy(rows_v.at[pl.ds(0, step)],
                            acc_sh.at[pl.ds(row0 + off, step)])
            off += step
        plsc.subcore_barrier()

        def body(i, carry):
            base = w * e_per_w + i * k
            pltpu.sync_copy(src_hbm.at[pl.ds(base, k)], src_v)
            pltpu.sync_copy(dst_hbm.at[pl.ds(base, k)], dst_v)
            pltpu.async_copy(table_hbm.at[src_v], rows_v, sem).wait()
            pltpu.sync_copy(rows_v, acc_sh.at[dst_v], add=True)
            return carry

        lax.fori_loop(0, n_chunks, body, 0)
        plsc.subcore_barrier()

        pltpu.sync_copy(acc_sh.at[pl.ds(row0, stripe)],
                        out_hbm.at[pl.ds(c * n_out + row0, stripe)])

    return fn


_N_OUT_NODE = 10112
_N_OUT_SEG = 384
_E_PAD = 327680


def _mlp1_body(x_ref, w_ref, b_ref, o_ref):
    o_ref[...] = jnp.maximum(
        jnp.dot(x_ref[...], w_ref[...], preferred_element_type=jnp.float32,
                precision=lax.Precision.HIGHEST)
        + b_ref[...], 0.0)


def _embed(x, w, b, n, bt):
    return pl.pallas_call(
        _mlp1_body,
        grid=(n // bt,),
        in_specs=[pl.BlockSpec((bt, x.shape[1]), lambda i: (i, 0)),
                  pl.BlockSpec(w.shape, lambda i: (0, 0)),
                  pl.BlockSpec((1, w.shape[1]), lambda i: (0, 0))],
        out_specs=pl.BlockSpec((bt, w.shape[1]), lambda i: (i, 0)),
        out_shape=jax.ShapeDtypeStruct((n, w.shape[1]), jnp.float32),
    )(x, w, b.reshape(1, -1))


def _solvent_body(x_ref, w1_ref, b1_ref, w2_ref, b2_ref, o_ref):
    h = jnp.maximum(
        jnp.dot(x_ref[...], w1_ref[...], preferred_element_type=jnp.float32,
                precision=lax.Precision.HIGHEST)
        + b1_ref[...], 0.0)
    o_ref[...] = (jnp.dot(h, w2_ref[...], preferred_element_type=jnp.float32,
                precision=lax.Precision.HIGHEST)
                  + b2_ref[...])


def _solvent_mlp(x, w1, b1, w2, b2):
    n = x.shape[0]
    bt = 1000
    return pl.pallas_call(
        _solvent_body,
        grid=(n // bt,),
        in_specs=[pl.BlockSpec((bt, x.shape[1]), lambda i: (i, 0)),
                  pl.BlockSpec(w1.shape, lambda i: (0, 0)),
                  pl.BlockSpec((1, w1.shape[1]), lambda i: (0, 0)),
                  pl.BlockSpec(w2.shape, lambda i: (0, 0)),
                  pl.BlockSpec((1, w2.shape[1]), lambda i: (0, 0))],
        out_specs=pl.BlockSpec((bt, w2.shape[1]), lambda i: (i, 0)),
        out_shape=jax.ShapeDtypeStruct((n, w2.shape[1]), jnp.float32),
    )(x, w1, b1.reshape(1, -1), w2, b2.reshape(1, -1))


_LBT = 400
_LGRID = _NSOLU // _LBT


def _layer_body(p0_ref, p1_ref, h_ref, c0_ref, c1_ref, et_ref, sr_ref,
                w1_ref, b1_ref, w2_ref, b2_ref, h2_ref, st_ref, acc):
    i = pl.program_id(0)
    cnt = c0_ref[...] + c1_ref[...]
    x = (p0_ref[...] + p1_ref[...] + h_ref[...] + sr_ref[...]
         + jnp.dot(cnt, et_ref[...], preferred_element_type=jnp.float32,
                precision=lax.Precision.HIGHEST))
    hmid = jnp.maximum(
        jnp.dot(x, w1_ref[...], preferred_element_type=jnp.float32,
                precision=lax.Precision.HIGHEST)
        + b1_ref[...], 0.0)
    h2 = (jnp.dot(hmid, w2_ref[...], preferred_element_type=jnp.float32,
                precision=lax.Precision.HIGHEST)
          + b2_ref[...])
    h2_ref[...] = h2

    @pl.when(i == 0)
    def _():
        acc[...] = jnp.zeros((8, _EMB), jnp.float32)

    acc[0:1, :] += jnp.sum(h2, axis=0, keepdims=True)
    acc[1:2, :] += jnp.sum(h2 * h2, axis=0, keepdims=True)

    @pl.when(i == _LGRID - 1)
    def _():
        st_ref[...] = acc[...]


def _gin_layer_mlp(p0, p1, h, c0, c1, et, sr, w1, b1, w2, b2):
    return pl.pallas_call(
        _layer_body,
        grid=(_LGRID,),
        in_specs=[pl.BlockSpec((_LBT, _EMB), lambda i: (i, 0)),
                  pl.BlockSpec((_LBT, _EMB), lambda i: (i, 0)),
                  pl.BlockSpec((_LBT, _EMB), lambda i: (i, 0)),
                  pl.BlockSpec((_LBT, _EMB), lambda i: (i, 0)),
                  pl.BlockSpec((_LBT, _EMB), lambda i: (i, 0)),
                  pl.BlockSpec((_EMB, _EMB), lambda i: (0, 0)),
                  pl.BlockSpec((1, _EMB), lambda i: (0, 0)),
                  pl.BlockSpec((_EMB, 2 * _EMB), lambda i: (0, 0)),
                  pl.BlockSpec((1, 2 * _EMB), lambda i: (0, 0)),
                  pl.BlockSpec((2 * _EMB, _EMB), lambda i: (0, 0)),
                  pl.BlockSpec((1, _EMB), lambda i: (0, 0))],
        out_specs=[pl.BlockSpec((_LBT, _EMB), lambda i: (i, 0)),
                   pl.BlockSpec((8, _EMB), lambda i: (0, 0))],
        out_shape=[jax.ShapeDtypeStruct((_NSOLU, _EMB), jnp.float32),
                   jax.ShapeDtypeStruct((8, _EMB), jnp.float32)],
        scratch_shapes=[pltpu.VMEM((8, _EMB), jnp.float32)],
    )(p0, p1, h, c0, c1, et, sr, w1, b1.reshape(1, -1), w2, b2.reshape(1, -1))


def _norm_body(relu, n_rows, h2_ref, st_ref, g_ref, bt_ref, o_ref):
    mu = st_ref[0:1, :] / n_rows
    var = st_ref[1:2, :] / n_rows - mu * mu
    y = (h2_ref[...] - mu) * lax.rsqrt(var + 1e-5) * g_ref[...] + bt_ref[...]
    if relu:
        y = jnp.maximum(y, 0.0)
    o_ref[...] = y


def _batchnorm(h2, st, gamma, beta, relu):
    return pl.pallas_call(
        functools.partial(_norm_body, relu, float(_NSOLU)),
        grid=(_LGRID,),
        in_specs=[pl.BlockSpec((_LBT, _EMB), lambda i: (i, 0)),
                  pl.BlockSpec((8, _EMB), lambda i: (0, 0)),
                  pl.BlockSpec((1, _EMB), lambda i: (0, 0)),
                  pl.BlockSpec((1, _EMB), lambda i: (0, 0))],
        out_specs=pl.BlockSpec((_LBT, _EMB), lambda i: (i, 0)),
        out_shape=jax.ShapeDtypeStruct((_NSOLU, _EMB), jnp.float32),
    )(h2, st, gamma.reshape(1, -1), beta.reshape(1, -1))


_IT = 400
_IGRID = _NSOLU // _IT
_JC = 1000
_NJC = _NSOLV // _JC


def _inter_body(u_ref, v_ref, wiu_ref, wiv_ref, bi_ref, u2_ref, v2_ref,
                vacc):
    i = pl.program_id(0)

    @pl.when(i == 0)
    def _():
        vacc[...] = jnp.zeros((_NSOLV, _EMB), jnp.float32)

    u_blk = u_ref[...]
    a = jnp.dot(u_blk, wiu_ref[...],
                preferred_element_type=jnp.float32,
                precision=lax.Precision.HIGHEST)[:, 0:1]
    bi = bi_ref[0:1, 0:1]
    u2acc = jnp.zeros((_IT, _EMB), jnp.float32)
    for j in range(_NJC):
        v_c = v_ref[pl.ds(j * _JC, _JC), :]
        bt = lax.dot_general(wiv_ref[...], v_c, (((0,), (1,)), ((), ())),
                             preferred_element_type=jnp.float32,
                precision=lax.Precision.HIGHEST)[0:1, :]
        t = jnp.tanh(a + bt + bi)
        u2acc += jnp.dot(t, v_c, preferred_element_type=jnp.float32,
                precision=lax.Precision.HIGHEST)
        vacc[pl.ds(j * _JC, _JC), :] += lax.dot_general(
            t, u_blk, (((0,), (0,)), ((), ())),
            preferred_element_type=jnp.float32,
                precision=lax.Precision.HIGHEST)

    u2_ref[...] = u_blk + u2acc

    @pl.when(i == _IGRID - 1)
    def _():
        v2_ref[...] = v_ref[...] + vacc[...]


def _interaction(u, v, wiu_p, wiv_p, bi_row):
    return pl.pallas_call(
        _inter_body,
        grid=(_IGRID,),
        in_specs=[pl.BlockSpec((_IT, _EMB), lambda i: (i, 0)),
                  pl.BlockSpec((_NSOLV, _EMB), lambda i: (0, 0)),
                  pl.BlockSpec((_EMB, 8), lambda i: (0, 0)),
                  pl.BlockSpec((_EMB, 8), lambda i: (0, 0)),
                  pl.BlockSpec((1, _EMB), lambda i: (0, 0))],
        out_specs=[pl.BlockSpec((_IT, _EMB), lambda i: (i, 0)),
                   pl.BlockSpec((_NSOLV, _EMB), lambda i: (0, 0))],
        out_shape=[jax.ShapeDtypeStruct((_NSOLU, _EMB), jnp.float32),
                   jax.ShapeDtypeStruct((_NSOLV, _EMB), jnp.float32)],
        scratch_shapes=[pltpu.VMEM((_NSOLV, _EMB), jnp.float32)],
    )(u, v, wiu_p, wiv_p, bi_row)


def _final_body(pu0_ref, pu1_ref, pv0_ref, pv1_ref, w0u_ref, w0v_ref,
                b0_ref, w1_ref, b1_ref, wl_ref, bl_ref, o_ref):
    pu = pu0_ref[...] + pu1_ref[...]
    pv = pv0_ref[...] + pv1_ref[...]
    g1 = jnp.maximum(
        jnp.dot(pu, w0u_ref[...], preferred_element_type=jnp.float32,
                precision=lax.Precision.HIGHEST)
        + jnp.dot(pv, w0v_ref[...], preferred_element_type=jnp.float32,
                precision=lax.Precision.HIGHEST)
        + b0_ref[...], 0.0)
    g2 = jnp.maximum(
        jnp.dot(g1, w1_ref[...], preferred_element_type=jnp.float32,
                precision=lax.Precision.HIGHEST)
        + b1_ref[...], 0.0)
    o_ref[...] = (jnp.dot(g2, wl_ref[...], preferred_element_type=jnp.float32,
                precision=lax.Precision.HIGHEST)
                  + bl_ref[...])


def _final_mlp(pu0, pu1, pv0, pv1, w0u, w0v, b0, w1, b1, wl_p, bl_row):
    full = lambda s: pl.BlockSpec(s, lambda: (0,) * len(s))
    return pl.pallas_call(
        _final_body,
        in_specs=[full((_NSEG, _EMB)), full((_NSEG, _EMB)),
                  full((_NSEG, _EMB)), full((_NSEG, _EMB)),
                  full((_EMB, _EMB)), full((_EMB, _EMB)),
                  full((1, _EMB)), full((_EMB, 64)), full((1, 64)),
                  full((64, _EMB)), full((1, _EMB))],
        out_specs=full((_NSEG, _EMB)),
        out_shape=jax.ShapeDtypeStruct((_NSEG, _EMB), jnp.float32),
    )(pu0, pu1, pv0, pv1, w0u, w0v, b0.reshape(1, -1), w1,
      b1.reshape(1, -1), wl_p, bl_row)


def kernel(solute_x, solute_edge_index, solute_edge_attr, solute_batch,
           solvent_x, solvent_batch, params):
    f32 = jnp.float32
    src = solute_edge_index[0]
    dst = solute_edge_index[1]

    pad_e = _E_PAD - _ESOLU
    src_p = jnp.concatenate([src, jnp.zeros((pad_e,), jnp.int32)])
    dst_p = jnp.concatenate([dst, jnp.full((pad_e,), _NSOLU, jnp.int32)])
    codes = solute_edge_attr[:, 0] * 3 + solute_edge_attr[:, 1]
    codes_p = jnp.concatenate([codes, jnp.zeros((pad_e,), jnp.int32)])

    iota_u = jnp.arange(_NSOLU, dtype=jnp.int32)
    pool_src_u = jnp.concatenate([iota_u, jnp.zeros((240,), jnp.int32)])
    pool_dst_u = jnp.concatenate(
        [solute_batch, jnp.full((240,), _NSEG, jnp.int32)])
    iota_v = jnp.arange(_NSOLV, dtype=jnp.int32)
    pool_src_v = jnp.concatenate([iota_v, jnp.zeros((120,), jnp.int32)])
    pool_dst_v = jnp.concatenate(
        [solvent_batch, jnp.full((120,), _NSEG, jnp.int32)])

    eye16 = jnp.eye(16, _EMB, dtype=f32)
    cnt_parts = _sc_segsum(16, _EMB, _E_PAD, 128, _N_OUT_NODE)(
        eye16, codes_p, dst_p)
    cnt0 = cnt_parts[:_NSOLU]
    cnt1 = cnt_parts[_N_OUT_NODE:_N_OUT_NODE + _NSOLU]

    h = _embed(solute_x, params['W_emb'], params['b_emb'], _NSOLU, _LBT)

    for l, p in enumerate(params['layers']):
        parts = _sc_segsum(_NSOLU, _EMB, _E_PAD, 128, _N_OUT_NODE)(
            h, src_p, dst_p)
        p0 = parts[:_NSOLU]
        p1 = parts[_N_OUT_NODE:_N_OUT_NODE + _NSOLU]
        et = (p['edge_emb1'][:3, None, :]
              + p['edge_emb2'][None, :3, :]).reshape(9, _EMB)
        et = jnp.concatenate([et, jnp.zeros((_EMB - 9, _EMB), f32)], axis=0)
        sr = (p['edge_emb1'][4] + p['edge_emb2'][0]).reshape(1, _EMB)
        h2, st = _gin_layer_mlp(p0, p1, h, cnt0, cnt1, et, sr,
                                p['W1'], p['b1'], p['W2'], p['b2'])
        h = _batchnorm(h2, st, p['gamma'], p['beta'],
                       relu=(l < len(params['layers']) - 1))

    u = h
    v = _solvent_mlp(solvent_x, params['Ws1'], params['bs1'],
                     params['Ws2'], params['bs2'])

    wiu_p = jnp.concatenate(
        [params['Wi_u'], jnp.zeros((_EMB, 7), f32)], axis=1)
    wiv_p = jnp.concatenate(
        [params['Wi_v'], jnp.zeros((_EMB, 7), f32)], axis=1)
    bi_row = jnp.broadcast_to(params['bi'], (1, _EMB)).astype(f32)
    u2, v2 = _interaction(u, v, wiu_p, wiv_p, bi_row)

    pu_parts = _sc_segsum(_NSOLU, _EMB, 10240, 80, _N_OUT_SEG)(
        u2, pool_src_u, pool_dst_u)
    pv_parts = _sc_segsum(_NSOLV, _EMB, 5120, 80, _N_OUT_SEG)(
        v2, pool_src_v, pool_dst_v)
    pu0 = pu_parts[:_NSEG]
    pu1 = pu_parts[_N_OUT_SEG:_N_OUT_SEG + _NSEG]
    pv0 = pv_parts[:_NSEG]
    pv1 = pv_parts[_N_OUT_SEG:_N_OUT_SEG + _NSEG]

    wl_p = jnp.concatenate(
        [params['Wlast'], jnp.zeros((64, _EMB - 1), f32)], axis=1)
    bl_row = jnp.concatenate(
        [params['blast'], jnp.zeros((_EMB - 1,), f32)]).reshape(1, _EMB)
    out = _final_mlp(pu0, pu1, pv0, pv1,
                     params['Wo0'][:_EMB], params['Wo0'][_EMB:],
                     params['bo0'], params['Wo1'], params['bo1'],
                     wl_p, bl_row)
    return out[:, 0:1]
